# Initial kernel scaffold; baseline (speedup 1.0000x reference)
import jax
import jax.numpy as jnp
from jax.experimental import pallas as pl


def kernel(x, edge_index, edge_attr, We1, be1, We2, be2, Wm1, bm1, Wm2, bm2, Wu1, bu1, Wu2, bu2, Wg, bg, ln_gamma, ln_beta):
    raise NotImplementedError("write your pallas kernel here")


# speedup vs baseline: 3.3652x; 3.3652x over previous
import functools
import jax
import jax.numpy as jnp
from jax import lax
from jax.experimental import pallas as pl
from jax.experimental.pallas import tpu as pltpu
from jax.experimental.pallas import tpu_sc as plsc

_BLK = 80
_NC = 2
_NS = 16


def _silu(v):
    return v * jax.nn.sigmoid(v)


def _dot(a, b):
    return jnp.dot(a, b, preferred_element_type=jnp.float32)


def _pre_body(x_ref, wa_ref, wb_ref, a_ref, b_ref):
    x = x_ref[...]
    a_ref[...] = _dot(x, wa_ref[...])
    b_ref[...] = _dot(x, wb_ref[...])


def _edge_body(ea_ref, pa_ref, pb_ref, we1_ref, be1_ref, we2_ref, be2_ref,
               w1c_ref, bm1_ref, wm2_ref, bm2_ref, wg_ref, bg_ref, msg_ref):
    h = _silu(_dot(ea_ref[...], we1_ref[...]) + be1_ref[...])
    emb = _dot(h, we2_ref[...]) + be2_ref[...]
    t = _silu(pa_ref[...] + pb_ref[...] + _dot(emb, w1c_ref[...]) + bm1_ref[...])
    g = jax.nn.sigmoid(_dot(emb, wg_ref[...]) + bg_ref[...])
    msg_ref[...] = (_dot(t, wm2_ref[...]) + bm2_ref[...]) * g


def _upd_body(p_ref, x_ref, wua_ref, wub_ref, bu1_ref, wu2_ref, bu2_ref,
              lng_ref, lnb_ref, out_ref):
    x = x_ref[...]
    n = x.shape[0]
    aggr = p_ref[0, :n] + p_ref[1, :n]
    u = _silu(_dot(aggr, wua_ref[...]) + _dot(x, wub_ref[...]) + bu1_ref[...])
    h = x + _dot(u, wu2_ref[...]) + bu2_ref[...]
    mu = jnp.mean(h, axis=-1, keepdims=True)
    d = h - mu
    var = jnp.mean(d * d, axis=-1, keepdims=True)
    out_ref[...] = d * lax.rsqrt(var + 1e-5) * lng_ref[...] + lnb_ref[...]


def _make_gather(E, D):
    nw = _NC * _NS
    epw = E // nw
    nblk = epw // _BLK
    mesh = plsc.VectorSubcoreMesh(core_axis_name="c", subcore_axis_name="s")

    @functools.partial(
        pl.kernel, mesh=mesh,
        out_type=(jax.ShapeDtypeStruct((E, D), jnp.float32),
                  jax.ShapeDtypeStruct((E, D), jnp.float32)),
        scratch_types=[
            pltpu.VMEM((nblk, _BLK), jnp.int32),
            pltpu.VMEM((nblk, _BLK), jnp.int32),
            pltpu.VMEM((_BLK, D), jnp.float32),
            pltpu.VMEM((_BLK, D), jnp.float32),
            pltpu.SemaphoreType.DMA,
            pltpu.SemaphoreType.DMA,
        ])
    def gather_k(a_hbm, b_hbm, dsti_hbm, srci_hbm, pa_hbm, pb_hbm,
                 di_v, si_v, ra_v, rb_v, sa, sb):
        c = lax.axis_index("c")
        s = lax.axis_index("s")
        wid = s * _NC + c
        pltpu.sync_copy(dsti_hbm.at[wid], di_v)
        pltpu.sync_copy(srci_hbm.at[wid], si_v)

        def body(j, carry):
            ca = pltpu.async_copy(a_hbm.at[di_v.at[j]], ra_v, sa)
            cb = pltpu.async_copy(b_hbm.at[si_v.at[j]], rb_v, sb)
            ca.wait()
            cb.wait()
            base = wid * epw + j * _BLK
            pltpu.sync_copy(ra_v, pa_hbm.at[pl.ds(base, _BLK)])
            pltpu.sync_copy(rb_v, pb_hbm.at[pl.ds(base, _BLK)])
            return carry

        lax.fori_loop(0, nblk, body, 0)

    return gather_k


def _make_scatter(E, N, D, npad):
    epc = E // _NC
    ept = epc // _NS
    nblk = ept // _BLK
    rpt = npad // _NS
    zrows = 80
    mesh = plsc.VectorSubcoreMesh(core_axis_name="c", subcore_axis_name="s")

    @functools.partial(
        pl.kernel, mesh=mesh,
        out_type=jax.ShapeDtypeStruct((_NC, npad, D), jnp.float32),
        scratch_types=[
            pltpu.VMEM((nblk, _BLK), jnp.int32),
            pltpu.VMEM((_BLK, D), jnp.float32),
            pltpu.VMEM((zrows, D), jnp.float32),
            pltpu.VMEM_SHARED((npad, D), jnp.float32),
        ])
    def scatter_k(msg_hbm, dsti_hbm, out_hbm, di_v, m_v, z_v, acc_sh):
        c = lax.axis_index("c")
        s = lax.axis_index("s")

        def zero_body(i, carry):
            for jj in range(D // 16):
                z_v[i, pl.ds(jj * 16, 16)] = jnp.zeros((16,), jnp.float32)
            return carry

        lax.fori_loop(0, zrows, zero_body, 0)
        for k in range(rpt // zrows):
            pltpu.sync_copy(z_v, acc_sh.at[pl.ds(s * rpt + k * zrows, zrows)])
        plsc.subcore_barrier()

        base_e = c * epc + s * ept
        pltpu.sync_copy(dsti_hbm.at[c * _NS + s], di_v)

        def body(j, carry):
            pltpu.sync_copy(msg_hbm.at[pl.ds(base_e + j * _BLK, _BLK)], m_v)
            pltpu.sync_copy(m_v, acc_sh.at[di_v.at[j]], add=True)
            return carry

        lax.fori_loop(0, nblk, body, 0)
        plsc.subcore_barrier()
        pltpu.sync_copy(acc_sh.at[pl.ds(s * rpt, rpt)],
                        out_hbm.at[c, pl.ds(s * rpt, rpt)])

    return scatter_k


def kernel(x, edge_index, edge_attr, We1, be1, We2, be2, Wm1, bm1, Wm2, bm2,
           Wu1, bu1, Wu2, bu2, Wg, bg, ln_gamma, ln_beta):
    N, D = x.shape
    E, R = edge_attr.shape
    f32 = jnp.float32

    nw = _NC * _NS
    src2 = edge_index[0].reshape(nw, E // (nw * _BLK), _BLK)
    dst2 = edge_index[1].reshape(nw, E // (nw * _BLK), _BLK)
    npad = ((N + nw * 8 - 1) // (nw * 8)) * nw * 8
    W1a, W1b, W1c = Wm1[:D], Wm1[D:2 * D], Wm1[2 * D:]
    Wua, Wub = Wu1[:D], Wu1[D:]
    row = lambda v: v.reshape(1, -1)

    A, B = pl.pallas_call(
        _pre_body,
        out_shape=(jax.ShapeDtypeStruct((N, D), f32),
                   jax.ShapeDtypeStruct((N, D), f32)),
    )(x, W1a, W1b)

    PA, PB = _make_gather(E, D)(A, B, dst2, src2)

    EB = 8000
    full = lambda w: pl.BlockSpec(w.shape, lambda i: (0,) * w.ndim)
    msg = pl.pallas_call(
        _edge_body,
        grid=(E // EB,),
        in_specs=[
            pl.BlockSpec((EB, R), lambda i: (i, 0)),
            pl.BlockSpec((EB, D), lambda i: (i, 0)),
            pl.BlockSpec((EB, D), lambda i: (i, 0)),
            full(We1), full(row(be1)), full(We2), full(row(be2)),
            full(W1c), full(row(bm1)), full(Wm2), full(row(bm2)),
            full(Wg), full(row(bg)),
        ],
        out_specs=pl.BlockSpec((EB, D), lambda i: (i, 0)),
        out_shape=jax.ShapeDtypeStruct((E, D), f32),
    )(edge_attr, PA, PB, We1, row(be1), We2, row(be2), W1c, row(bm1),
      Wm2, row(bm2), Wg, row(bg))

    partials = _make_scatter(E, N, D, npad)(msg, dst2)

    out = pl.pallas_call(
        _upd_body,
        out_shape=jax.ShapeDtypeStruct((N, D), f32),
    )(partials, x, Wua, Wub, row(bu1), Wu2, row(bu2),
      row(ln_gamma), row(ln_beta))
    return out
